# baseline (speedup 1.0000x reference)
import jax
import jax.numpy as jnp
from jax.experimental import pallas as pl
from jax.experimental.pallas import tpu as pltpu


def _se_block(x_ref, w1_ref, b1_ref, w2_ref, b2_ref, o_ref, *, inv_hw):
    x = x_ref[...]
    w1t = jnp.transpose(w1_ref[...]) * inv_hw
    w2t = jnp.transpose(w2_ref[...])
    y = jnp.sum(x, axis=2, dtype=jnp.float32)
    h = jnp.dot(y, w1t, preferred_element_type=jnp.float32) + b1_ref[...]
    h = jnp.maximum(h, 0.0) + 0.2 * jnp.minimum(h, 0.0)
    g = jnp.dot(h, w2t, preferred_element_type=jnp.float32) + b2_ref[...]
    g = jnp.tanh(g)
    o_ref[...] = x * g[:, :, None]


def kernel(x, w1, b1, w2, b2):
    B, C, H, W = x.shape
    Cr = w1.shape[0]
    HW = H * W
    x3 = x.reshape(B, C, HW)
    b1r = b1.reshape(1, Cr)
    b2r = b2.reshape(1, C)

    bt = 22
    import functools
    body = functools.partial(_se_block, inv_hw=1.0 / HW)
    out = pl.pallas_call(
        body,
        out_shape=jax.ShapeDtypeStruct((B, C, HW), x3.dtype),
        grid=(pl.cdiv(B, bt),),
        in_specs=[
            pl.BlockSpec((bt, C, HW), lambda b: (b, 0, 0)),
            pl.BlockSpec((Cr, C), lambda b: (0, 0)),
            pl.BlockSpec((1, Cr), lambda b: (0, 0)),
            pl.BlockSpec((C, Cr), lambda b: (0, 0)),
            pl.BlockSpec((1, C), lambda b: (0, 0)),
        ],
        out_specs=pl.BlockSpec((bt, C, HW), lambda b: (b, 0, 0)),
        compiler_params=pltpu.CompilerParams(
            dimension_semantics=("parallel",),
            vmem_limit_bytes=48 * 1024 * 1024,
        ),
    )(x3, w1, b1r, w2, b2r)
    return out.reshape(B, C, H, W)


# speedup vs baseline: 1.0023x; 1.0023x over previous
import jax
import jax.numpy as jnp
from jax.experimental import pallas as pl
from jax.experimental.pallas import tpu as pltpu


def _se_block(x_ref, w1_ref, b1_ref, w2_ref, b2_ref, o_ref, *, inv_hw):
    x = x_ref[...]
    w1t = jnp.transpose(w1_ref[...]) * inv_hw
    w2t = jnp.transpose(w2_ref[...])
    y = jnp.sum(x, axis=2, dtype=jnp.float32)
    h = jnp.dot(y, w1t, preferred_element_type=jnp.float32) + b1_ref[...]
    h = jnp.maximum(h, 0.0) + 0.2 * jnp.minimum(h, 0.0)
    g = jnp.dot(h, w2t, preferred_element_type=jnp.float32) + b2_ref[...]
    g = jnp.tanh(g)
    o_ref[...] = x * g[:, :, None]


def kernel(x, w1, b1, w2, b2):
    B, C, H, W = x.shape
    Cr = w1.shape[0]
    HW = H * W
    x3 = x.reshape(B, C, HW)
    b1r = b1.reshape(1, Cr)
    b2r = b2.reshape(1, C)

    bt = 28
    import functools
    body = functools.partial(_se_block, inv_hw=1.0 / HW)
    out = pl.pallas_call(
        body,
        out_shape=jax.ShapeDtypeStruct((B, C, HW), x3.dtype),
        grid=(pl.cdiv(B, bt),),
        in_specs=[
            pl.BlockSpec((bt, C, HW), lambda b: (b, 0, 0)),
            pl.BlockSpec((Cr, C), lambda b: (0, 0)),
            pl.BlockSpec((1, Cr), lambda b: (0, 0)),
            pl.BlockSpec((C, Cr), lambda b: (0, 0)),
            pl.BlockSpec((1, C), lambda b: (0, 0)),
        ],
        out_specs=pl.BlockSpec((bt, C, HW), lambda b: (b, 0, 0)),
        compiler_params=pltpu.CompilerParams(
            dimension_semantics=("parallel",),
            vmem_limit_bytes=48 * 1024 * 1024,
        ),
    )(x3, w1, b1r, w2, b2r)
    return out.reshape(B, C, H, W)


# speedup vs baseline: 1.0035x; 1.0012x over previous
import functools

import jax
import jax.numpy as jnp
from jax.experimental import pallas as pl
from jax.experimental.pallas import tpu as pltpu


def _se_block(x_ref, w1_ref, b1_ref, w2_ref, b2_ref, o_ref, *, inv_hw):
    x = x_ref[...]
    w1t = jnp.transpose(w1_ref[...]) * inv_hw
    w2t = jnp.transpose(w2_ref[...])
    y = jnp.sum(x, axis=2, dtype=jnp.float32)
    h = jnp.dot(y, w1t, preferred_element_type=jnp.float32) + b1_ref[...]
    h = jnp.maximum(h, 0.0) + 0.2 * jnp.minimum(h, 0.0)
    g = jnp.dot(h, w2t, preferred_element_type=jnp.float32) + b2_ref[...]
    g = jnp.tanh(g)
    o_ref[...] = x * g[:, :, None]


def kernel(x, w1, b1, w2, b2):
    B, C, H, W = x.shape
    Cr = w1.shape[0]
    HW = H * W
    x3 = x.reshape(B, C, HW)
    b1r = b1.reshape(1, Cr)
    b2r = b2.reshape(1, C)

    bt = 28
    body = functools.partial(_se_block, inv_hw=1.0 / HW)
    out = pl.pallas_call(
        body,
        out_shape=jax.ShapeDtypeStruct((B, C, HW), x3.dtype),
        grid=(pl.cdiv(B, bt),),
        in_specs=[
            pl.BlockSpec((bt, C, HW), lambda b: (b, 0, 0)),
            pl.BlockSpec((Cr, C), lambda b: (0, 0)),
            pl.BlockSpec((1, Cr), lambda b: (0, 0)),
            pl.BlockSpec((C, Cr), lambda b: (0, 0)),
            pl.BlockSpec((1, C), lambda b: (0, 0)),
        ],
        out_specs=pl.BlockSpec((bt, C, HW), lambda b: (b, 0, 0)),
        compiler_params=pltpu.CompilerParams(
            dimension_semantics=("parallel",),
            vmem_limit_bytes=48 * 1024 * 1024,
        ),
    )(x3, w1, b1r, w2, b2r)
    return out.reshape(B, C, H, W)
